# baseline (speedup 1.0000x reference)
import jax
import jax.numpy as jnp
from jax import lax
from jax.experimental import pallas as pl
from jax.experimental.pallas import tpu as pltpu
from jax.experimental.pallas import tpu_sc as plsc

BATCH = 1024
MAXLEN = 200
N_FEAT = 64
CHUNK = 64
N_FLAT = BATCH * MAXLEN
N_CHUNKS = N_FLAT // CHUNK
NBUF = 5
DEPTH = 4


def _emb_body(x_hbm, pe_hbm, E2_hbm, out_hbm, idx_v, rows_v, out_v, pe_v,
              sems_g, sems_w):
    info = plsc.get_sparse_core_info()
    nc, ns = info.num_cores, info.num_subcores
    nw = nc * ns
    wid = lax.axis_index("s") * nc + lax.axis_index("c")
    chunks_per_w = N_CHUNKS // nw
    lbase = wid * chunks_per_w * CHUNK

    pltpu.sync_copy(pe_hbm, pe_v)
    pltpu.sync_copy(x_hbm.at[pl.ds(lbase, chunks_per_w * CHUNK)], idx_v)

    def idx16_of(c, k):
        return idx_v[pl.ds(c * CHUNK + k * 16, 16)]

    def g_copies(c, u):
        cps = []
        for k in range(CHUNK // 16):
            idxh = lax.shift_right_logical(idx16_of(c, k), 1)
            cps.append(pltpu.make_async_copy(
                E2_hbm.at[idxh], rows_v.at[u].at[pl.ds(k * 16, 16)],
                sems_g[u]))
        return cps

    def w_copy(c, u):
        return pltpu.make_async_copy(
            out_v.at[u],
            out_hbm.at[pl.ds(lbase // 128 * 64 + c * (CHUNK // 2),
                             CHUNK // 2)],
            sems_w[u])

    def g_start(c, u):
        for cp in g_copies(c, u):
            cp.start()

    for d in range(DEPTH):
        g_start(d, d)

    def group(g, carry):
        for u in range(NBUF):
            c = g * NBUF + u
            nxt = (u + DEPTH) % NBUF

            @pl.when(c + DEPTH < chunks_per_w)
            def _():
                g_start(c + DEPTH, nxt)

            for cp in g_copies(c, u):
                cp.wait()

            @pl.when(c >= NBUF)
            def _():
                w_copy(c - NBUF, u).wait()

            rows_b = rows_v.at[u]
            out_b = out_v.at[u]
            t0 = lax.rem(c * CHUNK, MAXLEN)

            @plsc.parallel_loop(0, CHUNK // 16, step=1)
            def _(k):
                off16 = (idx16_of(c, k) & 1) * N_FEAT
                for j in range(16):
                    r = k * 16 + j
                    t = t0 + r
                    t = jnp.where(t >= MAXLEN, t - MAXLEN, t)
                    th, tl = lax.shift_right_logical(t, 1), lax.rem(t, 2)
                    off = off16[j]
                    orow = k * 8 + j // 2
                    for q in range(N_FEAT // 16):
                        out_b[orow,
                              pl.ds((j % 2) * N_FEAT + q * 16, 16)] = (
                            rows_b[r, pl.ds(off + q * 16, 16)]
                            + pe_v[th, pl.ds(tl * N_FEAT + q * 16, 16)])

            w_copy(c, u).start()
        return carry

    lax.fori_loop(0, chunks_per_w // NBUF, group, 0)

    for u in range(NBUF):
        c = chunks_per_w - NBUF + u
        w_copy(c, c % NBUF).wait()


def kernel(x, E, pe):
    pe2 = pe.reshape(MAXLEN // 2, 2 * N_FEAT)
    x3 = x.reshape(N_FLAT)
    E2 = E.reshape(E.shape[0] // 2, 2 * N_FEAT)
    mesh = plsc.VectorSubcoreMesh(core_axis_name="c", subcore_axis_name="s")
    f = pl.kernel(
        _emb_body,
        out_type=jax.ShapeDtypeStruct((N_FLAT * N_FEAT // 128, 128),
                                      jnp.float32),
        mesh=mesh,
        compiler_params=pltpu.CompilerParams(use_tc_tiling_on_sc=True),
        scratch_types=[
            pltpu.VMEM((N_FLAT // 32,), jnp.int32),
            pltpu.VMEM((NBUF, CHUNK, 2 * N_FEAT), jnp.float32),
            pltpu.VMEM((NBUF, CHUNK // 2, 128), jnp.float32),
            pltpu.VMEM((MAXLEN // 2, 2 * N_FEAT), jnp.float32),
            [pltpu.SemaphoreType.DMA] * NBUF,
            [pltpu.SemaphoreType.DMA] * NBUF,
        ],
    )
    out = f(x3, pe2, E2)
    return out.reshape(BATCH, MAXLEN, N_FEAT)


# speedup vs baseline: 1.2247x; 1.2247x over previous
import jax
import jax.numpy as jnp
from jax import lax
from jax.experimental import pallas as pl
from jax.experimental.pallas import tpu as pltpu
from jax.experimental.pallas import tpu_sc as plsc

BATCH = 1024
MAXLEN = 200
N_FEAT = 64
CHUNK = 64
N_FLAT = BATCH * MAXLEN
N_CHUNKS = N_FLAT // CHUNK
NBUF = 5
DEPTH = 4


def _emb_body(x_hbm, pe_hbm, E2_hbm, out_hbm, idx_v, rows_v, out_v, pe_v,
              sems_g, sems_w):
    info = plsc.get_sparse_core_info()
    nc, ns = info.num_cores, info.num_subcores
    nw = nc * ns
    wid = lax.axis_index("s") * nc + lax.axis_index("c")
    chunks_per_w = N_CHUNKS // nw
    lbase = wid * chunks_per_w * CHUNK

    pltpu.sync_copy(pe_hbm, pe_v)
    pltpu.sync_copy(x_hbm.at[pl.ds(lbase, chunks_per_w * CHUNK)], idx_v)

    def idx16_of(c, k):
        return idx_v[pl.ds(c * CHUNK + k * 16, 16)]

    def g_copies(c, u):
        cps = []
        for k in range(CHUNK // 16):
            cps.append(pltpu.make_async_copy(
                E2_hbm.at[idx16_of(c, k)], rows_v.at[u].at[pl.ds(k * 16, 16)],
                sems_g[u]))
        return cps

    def w_copy(c, u):
        return pltpu.make_async_copy(
            out_v.at[u],
            out_hbm.at[pl.ds(lbase // 128 * 64 + c * (CHUNK // 2),
                             CHUNK // 2)],
            sems_w[u])

    def g_start(c, u):
        for cp in g_copies(c, u):
            cp.start()

    for d in range(DEPTH):
        g_start(d, d)

    def group(g, carry):
        for u in range(NBUF):
            c = g * NBUF + u
            nxt = (u + DEPTH) % NBUF

            @pl.when(c + DEPTH < chunks_per_w)
            def _():
                g_start(c + DEPTH, nxt)

            for cp in g_copies(c, u):
                cp.wait()

            @pl.when(c >= NBUF)
            def _():
                w_copy(c - NBUF, u).wait()

            rows_b = rows_v.at[u]
            out_b = out_v.at[u]
            t0 = lax.rem(c * CHUNK, MAXLEN)

            @plsc.parallel_loop(0, CHUNK, step=1, unroll=4)
            def _(r):
                t = t0 + r
                t = jnp.where(t >= MAXLEN, t - MAXLEN, t)
                th = lax.shift_right_logical(t, 1)
                tl = lax.rem(t, 2)
                orow = lax.shift_right_logical(r, 1)
                ocol = lax.rem(r, 2) * N_FEAT
                for q in range(N_FEAT // 16):
                    out_b[orow, pl.ds(ocol + q * 16, 16)] = (
                        rows_b[r, pl.ds(q * 16, 16)]
                        + pe_v[th, pl.ds(tl * N_FEAT + q * 16, 16)])

            w_copy(c, u).start()
        return carry

    lax.fori_loop(0, chunks_per_w // NBUF, group, 0)

    for u in range(NBUF):
        c = chunks_per_w - NBUF + u
        w_copy(c, c % NBUF).wait()


def kernel(x, E, pe):
    pe2 = pe.reshape(MAXLEN // 2, 2 * N_FEAT)
    x3 = x.reshape(N_FLAT)
    E2 = jnp.pad(E, ((0, 0), (0, N_FEAT)))
    mesh = plsc.VectorSubcoreMesh(core_axis_name="c", subcore_axis_name="s")
    f = pl.kernel(
        _emb_body,
        out_type=jax.ShapeDtypeStruct((N_FLAT * N_FEAT // 128, 128),
                                      jnp.float32),
        mesh=mesh,
        compiler_params=pltpu.CompilerParams(use_tc_tiling_on_sc=True),
        scratch_types=[
            pltpu.VMEM((N_FLAT // 32,), jnp.int32),
            pltpu.VMEM((NBUF, CHUNK, 128), jnp.float32),
            pltpu.VMEM((NBUF, CHUNK // 2, 128), jnp.float32),
            pltpu.VMEM((MAXLEN // 2, 2 * N_FEAT), jnp.float32),
            [pltpu.SemaphoreType.DMA] * NBUF,
            [pltpu.SemaphoreType.DMA] * NBUF,
        ],
    )
    out = f(x3, pe2, E2)
    return out.reshape(BATCH, MAXLEN, N_FEAT)


# speedup vs baseline: 1.2379x; 1.0108x over previous
import jax
import jax.numpy as jnp
from jax import lax
from jax.experimental import pallas as pl
from jax.experimental.pallas import tpu as pltpu
from jax.experimental.pallas import tpu_sc as plsc

BATCH = 1024
MAXLEN = 200
N_FEAT = 64
BB = 128
N_FLAT = BATCH * MAXLEN
N_UNITS = N_FLAT // BB
NBUF = 2


def _emb_body(x_hbm, pe_hbm, E2_hbm, out_hbm, idx_v, rows_v, out_v, pe_v,
              sems_g, sems_w):
    info = plsc.get_sparse_core_info()
    nc, ns = info.num_cores, info.num_subcores
    nw = nc * ns
    wid = lax.axis_index("s") * nc + lax.axis_index("c")
    units_per_w = N_UNITS // nw
    ubase = wid * units_per_w

    pltpu.sync_copy(pe_hbm, pe_v)
    pltpu.sync_copy(x_hbm.at[pl.ds(ubase * BB, units_per_w * BB)], idx_v)

    def g_copies(c, u):
        cps = []
        for k in range(BB // 16):
            idx16 = idx_v[pl.ds(c * BB + k * 16, 16)]
            cps.append(pltpu.make_async_copy(
                E2_hbm.at[idx16], rows_v.at[u].at[pl.ds(k * 16, 16)],
                sems_g[u]))
        return cps

    def w_copy(c, u):
        g = ubase + c
        t = lax.shift_right_logical(g, 3)
        bb = lax.rem(g, 8)
        return pltpu.make_async_copy(
            out_v.at[u], out_hbm.at[t].at[:, pl.ds(bb * BB, BB)], sems_w[u])

    def g_start(c, u):
        for cp in g_copies(c, u):
            cp.start()

    def compute(c, u):
        g = ubase + c
        t = lax.shift_right_logical(g, 3)
        th = lax.shift_right_logical(t, 1)
        tlo = lax.rem(t, 2) * N_FEAT
        rows_b = rows_v.at[u]
        out_b = out_v.at[u]

        @plsc.parallel_loop(0, N_FEAT, step=1, unroll=2)
        def _(f):
            pe16 = plsc.load_gather(
                pe_v, [jnp.full((16,), th, jnp.int32),
                       jnp.full((16,), tlo + f, jnp.int32)])
            cols = jnp.full((16,), f, jnp.int32)
            for lg in range(BB // 16):
                rows16 = lax.iota(jnp.int32, 16) + lg * 16
                out_b[f, pl.ds(lg * 16, 16)] = (
                    plsc.load_gather(rows_b, [rows16, cols]) + pe16)

    g_start(0, 0)

    def pair(p, carry):
        for u in range(NBUF):
            c = p * NBUF + u

            @pl.when(c + 1 < units_per_w)
            def _():
                g_start(c + 1, (u + 1) % NBUF)

            for cp in g_copies(c, u):
                cp.wait()

            @pl.when(c >= NBUF)
            def _():
                w_copy(c - NBUF, u).wait()

            compute(c, u)
            w_copy(c, u).start()
        return carry

    lax.fori_loop(0, units_per_w // NBUF, pair, 0)

    for u in range(NBUF):
        c = units_per_w - NBUF + u
        w_copy(c, c % NBUF).wait()


def kernel(x, E, pe):
    pe2 = pe.reshape(MAXLEN // 2, 2 * N_FEAT)
    xf = jnp.transpose(x).reshape(N_FLAT)
    E2 = jnp.pad(E, ((0, 0), (0, N_FEAT)))
    mesh = plsc.VectorSubcoreMesh(core_axis_name="c", subcore_axis_name="s")
    f = pl.kernel(
        _emb_body,
        out_type=jax.ShapeDtypeStruct((MAXLEN, N_FEAT, BATCH), jnp.float32),
        mesh=mesh,
        compiler_params=pltpu.CompilerParams(use_tc_tiling_on_sc=True,
                                             needs_layout_passes=False),
        scratch_types=[
            pltpu.VMEM((N_FLAT // 32,), jnp.int32),
            pltpu.VMEM((NBUF, BB, 128), jnp.float32),
            pltpu.VMEM((NBUF, N_FEAT, BB), jnp.float32),
            pltpu.VMEM((MAXLEN // 2, 2 * N_FEAT), jnp.float32),
            [pltpu.SemaphoreType.DMA] * NBUF,
            [pltpu.SemaphoreType.DMA] * NBUF,
        ],
    )
    out = f(xf, pe2, E2)
    return jnp.transpose(out, (2, 0, 1))
